# Initial kernel scaffold; baseline (speedup 1.0000x reference)
import jax
import jax.numpy as jnp
from jax.experimental import pallas as pl


def kernel(indices, embedding):
    raise NotImplementedError("write your pallas kernel here")


# speedup vs baseline: 2.7517x; 2.7517x over previous
import functools

import jax
import jax.numpy as jnp
from jax import lax
from jax.experimental import pallas as pl
from jax.experimental.pallas import tpu as pltpu
from jax.experimental.pallas import tpu_sc as plsc

D_MODEL = 128
CHUNK = 128


@functools.lru_cache(maxsize=None)
def _build(n_idx: int):
    info = plsc.get_sparse_core_info()
    nc, ns = info.num_cores, info.num_subcores
    nw = nc * ns
    n_chunks = n_idx // CHUNK
    chunks_per_w = n_chunks // nw
    assert n_chunks % nw == 0 and n_idx % CHUNK == 0

    mesh = plsc.VectorSubcoreMesh(core_axis_name="c", subcore_axis_name="s")

    @functools.partial(
        pl.kernel,
        mesh=mesh,
        out_type=jax.ShapeDtypeStruct((nw, chunks_per_w, CHUNK, D_MODEL), jnp.float32),
        scratch_types=[
            pltpu.VMEM((chunks_per_w, CHUNK), jnp.int32),
            pltpu.VMEM((CHUNK, D_MODEL), jnp.float32),
            pltpu.SemaphoreType.DMA,
        ],
    )
    def k(idx_hbm, table_hbm, out_hbm, idx_v, rows_v, sem):
        wid = lax.axis_index("s") * nc + lax.axis_index("c")
        pltpu.sync_copy(idx_hbm.at[wid], idx_v)

        def step(j, carry):
            pltpu.async_copy(table_hbm.at[idx_v.at[j]], rows_v, sem).wait()
            pltpu.sync_copy(rows_v, out_hbm.at[wid, j])
            return carry

        lax.fori_loop(0, chunks_per_w, step, 0)

    return k


def kernel(indices, embedding):
    b, s = indices.shape
    n = b * s
    info = plsc.get_sparse_core_info()
    nw = info.num_cores * info.num_subcores
    idx = indices.reshape(nw, n // (nw * CHUNK), CHUNK).astype(jnp.int32)
    out = _build(n)(idx, embedding)
    return out.reshape(b, s, D_MODEL)


# speedup vs baseline: 2.8907x; 1.0505x over previous
import functools

import jax
import jax.numpy as jnp
from jax import lax
from jax.experimental import pallas as pl
from jax.experimental.pallas import tpu as pltpu
from jax.experimental.pallas import tpu_sc as plsc

D_MODEL = 128
CHUNK = 128


@functools.lru_cache(maxsize=None)
def _build(n_idx: int):
    info = plsc.get_sparse_core_info()
    nc, ns = info.num_cores, info.num_subcores
    nw = nc * ns
    n_chunks = n_idx // CHUNK
    chunks_per_w = n_chunks // nw
    assert n_chunks % nw == 0 and n_idx % CHUNK == 0

    mesh = plsc.VectorSubcoreMesh(core_axis_name="c", subcore_axis_name="s")

    nbuf = 5
    assert chunks_per_w % nbuf == 0
    n_groups = chunks_per_w // nbuf

    @functools.partial(
        pl.kernel,
        mesh=mesh,
        out_type=jax.ShapeDtypeStruct((nw, chunks_per_w, CHUNK, D_MODEL), jnp.float32),
        scratch_types=[
            pltpu.VMEM((chunks_per_w, CHUNK), jnp.int32),
            pltpu.VMEM((nbuf, CHUNK, D_MODEL), jnp.float32),
        ]
        + [pltpu.SemaphoreType.DMA] * (2 * nbuf),
    )
    def k(idx_hbm, table_hbm, out_hbm, idx_v, rows_v, *sems):
        gsems, wsems = sems[:nbuf], sems[nbuf:]
        wid = lax.axis_index("s") * nc + lax.axis_index("c")
        pltpu.sync_copy(idx_hbm.at[wid], idx_v)

        for b in range(nbuf):
            pltpu.async_copy(table_hbm.at[idx_v.at[b]], rows_v.at[b], gsems[b])

        def group(g, carry):
            j0 = g * nbuf
            for b in range(nbuf):
                pltpu.make_async_copy(
                    table_hbm.at[idx_v.at[j0 + b]], rows_v.at[b], gsems[b]
                ).wait()
                pltpu.async_copy(rows_v.at[b], out_hbm.at[wid, j0 + b], wsems[b])
            for b in range(nbuf):
                pltpu.make_async_copy(
                    rows_v.at[b], out_hbm.at[wid, j0 + b], wsems[b]
                ).wait()

                @pl.when(g + 1 < n_groups)
                def _():
                    pltpu.async_copy(
                        table_hbm.at[idx_v.at[j0 + nbuf + b]], rows_v.at[b], gsems[b]
                    )

            return carry

        lax.fori_loop(0, n_groups, group, 0)

    return k


def kernel(indices, embedding):
    b, s = indices.shape
    n = b * s
    info = plsc.get_sparse_core_info()
    nw = info.num_cores * info.num_subcores
    idx = indices.reshape(nw, n // (nw * CHUNK), CHUNK).astype(jnp.int32)
    out = _build(n)(idx, embedding)
    return out.reshape(b, s, D_MODEL)


# speedup vs baseline: 4.7018x; 1.6265x over previous
import functools

import jax
import jax.numpy as jnp
from jax import lax
from jax.experimental import pallas as pl
from jax.experimental.pallas import tpu as pltpu
from jax.experimental.pallas import tpu_sc as plsc

D_MODEL = 128
NB = 8


@functools.lru_cache(maxsize=None)
def _build(n_batch: int, seq: int):
    info = plsc.get_sparse_core_info()
    nc, ns = info.num_cores, info.num_subcores
    nw = nc * ns
    bat_per_w = n_batch // nw
    n_blocks = bat_per_w // NB
    assert n_batch % nw == 0 and bat_per_w % NB == 0 and n_blocks % 2 == 0

    mesh = plsc.VectorSubcoreMesh(core_axis_name="c", subcore_axis_name="s")

    @functools.partial(
        pl.kernel,
        mesh=mesh,
        out_type=jax.ShapeDtypeStruct((n_batch, seq, D_MODEL), jnp.float32),
        scratch_types=[
            pltpu.VMEM((bat_per_w, seq), jnp.int32),
            pltpu.VMEM((2, NB, seq, D_MODEL), jnp.float32),
            pltpu.SemaphoreType.DMA,
            pltpu.SemaphoreType.DMA,
            pltpu.SemaphoreType.DMA,
            pltpu.SemaphoreType.DMA,
        ],
    )
    def k(idx_hbm, table_hbm, out_hbm, idx_v, rows_v, g0, g1, w0, w1):
        gsems = (g0, g1)
        wsems = (w0, w1)
        wid = lax.axis_index("s") * nc + lax.axis_index("c")
        base = wid * bat_per_w
        pltpu.sync_copy(idx_hbm.at[pl.ds(base, bat_per_w)], idx_v)

        def fire_block(kb, bb):
            for i in range(NB):
                pltpu.async_copy(
                    table_hbm.at[idx_v.at[kb * NB + i]],
                    rows_v.at[bb, i],
                    gsems[bb],
                )

        def drain_block(bb):
            pltpu.make_async_copy(
                out_hbm.at[pl.ds(base, NB)], rows_v.at[bb], gsems[bb]
            ).wait()

        fire_block(0, 0)

        def group(g, carry):
            for u in range(2):
                kb = g * 2 + u
                bb = u
                nxt = 1 - u

                @pl.when(kb + 1 < n_blocks)
                def _():
                    @pl.when(kb >= 1)
                    def _():
                        pltpu.make_async_copy(
                            rows_v.at[nxt],
                            out_hbm.at[pl.ds(base, NB)],
                            wsems[nxt],
                        ).wait()

                    fire_block(kb + 1, nxt)

                drain_block(bb)
                pltpu.async_copy(
                    rows_v.at[bb],
                    out_hbm.at[pl.ds(base + kb * NB, NB)],
                    wsems[bb],
                )
            return carry

        lax.fori_loop(0, n_blocks // 2, group, 0)

        for bb in range(2):
            pltpu.make_async_copy(
                rows_v.at[bb], out_hbm.at[pl.ds(base, NB)], wsems[bb]
            ).wait()

    return k


def kernel(indices, embedding):
    b, s = indices.shape
    out = _build(b, s)(indices.astype(jnp.int32), embedding)
    return out
